# Initial kernel scaffold; baseline (speedup 1.0000x reference)
import jax
import jax.numpy as jnp
from jax.experimental import pallas as pl


def kernel(x_all, a_cmp, mask, invn, pool_sel, w_slab, b_slab):
    raise NotImplementedError("write your pallas kernel here")


# speedup vs baseline: 1.0378x; 1.0378x over previous
import jax
import jax.numpy as jnp
from jax import lax
from jax.experimental import pallas as pl
from jax.experimental.pallas import tpu as pltpu

MAX_NUMS = 16
NN = 2 * MAX_NUMS
D_IN = 8
HIDDEN = 32
PAIRS = 8
TILE_ROWS = PAIRS * NN
POOL_ROWS = 2 * PAIRS

_W0_OFF = 0
_W1_OFF = _W0_OFF + D_IN
_WF0A_OFF = _W1_OFF + HIDDEN
_WF0B_OFF = _WF0A_OFF + HIDDEN
_WF1_OFF = _WF0B_OFF + HIDDEN
_W_ROWS = 112
_B_ROWS = 8


def _body(x_ref, a_ref, mask_ref, invn_ref, pool_ref, w_ref, b_ref,
          tile32_ref, bmask_ref, out_ref, xw_sc, p1_sc, p2_sc):
    T = a_ref.shape[0]
    TP = p1_sc.shape[0]

    w = w_ref[...]
    w1 = w[_W1_OFF:_W1_OFF + HIDDEN, :]
    wf0a = w[_WF0A_OFF:_WF0A_OFF + HIDDEN, :]
    wf0b = w[_WF0B_OFF:_WF0B_OFF + HIDDEN, :]
    wf1r8 = w[_WF1_OFF:_WF1_OFF + 8, :]
    w0 = w[_W0_OFF:_W0_OFF + D_IN, :]
    b0 = b_ref[0:1, :]
    b1 = b_ref[1:2, :]
    bf0 = b_ref[2:3, :]
    bf1 = b_ref[3:4, 0:1]
    pool_sel = pool_ref[...]
    tile32 = tile32_ref[...]
    bmask = bmask_ref[...]

    x_flat = x_ref[...].reshape(T * TILE_ROWS, D_IN)
    xw_sc[...] = jnp.dot(x_flat, w0,
                         preferred_element_type=jnp.float32).astype(jnp.bfloat16)

    def tile_body(t, carry):
        a_rows = a_ref[t].reshape(TILE_ROWS, NN)
        bd = jnp.dot(a_rows, tile32,
                     preferred_element_type=jnp.float32)
        bd = bd.astype(jnp.bfloat16) * bmask

        r0 = pl.multiple_of(t * TILE_ROWS, TILE_ROWS)
        h = jnp.dot(bd, xw_sc[pl.ds(r0, TILE_ROWS), :],
                    preferred_element_type=jnp.float32) + b0
        h = jnp.maximum(h, 0.0)

        agg = jnp.dot(bd, h.astype(jnp.bfloat16),
                      preferred_element_type=jnp.float32)
        h = jnp.dot(agg.astype(jnp.bfloat16), w1,
                    preferred_element_type=jnp.float32) + b1
        h = jnp.maximum(h, 0.0)

        hm = (h * mask_ref[t]).astype(jnp.bfloat16)
        pooled = jnp.dot(pool_sel, hm,
                         preferred_element_type=jnp.float32)
        pooled = pooled * invn_ref[t]

        p0 = pl.multiple_of(t * PAIRS, PAIRS)
        p1_sc[pl.ds(p0, PAIRS), :] = pooled[:PAIRS, :]
        p2_sc[pl.ds(p0, PAIRS), :] = pooled[PAIRS:, :]
        return carry

    lax.fori_loop(0, T, tile_body, 0)

    hg1 = p1_sc[...].astype(jnp.bfloat16)
    hg2 = p2_sc[...].astype(jnp.bfloat16)
    z = (jnp.dot(hg1, wf0a, preferred_element_type=jnp.float32)
         + jnp.dot(hg2, wf0b, preferred_element_type=jnp.float32) + bf0)
    z = jnp.maximum(z, 0.0)
    logit8 = lax.dot_general(wf1r8, z.astype(jnp.bfloat16),
                             (((1,), (1,)), ((), ())),
                             preferred_element_type=jnp.float32)
    out_ref[...] = jax.nn.sigmoid(logit8 + bf1).reshape(1, 8, TP)


def _forward(x_all, a_cmp, mask, invn, pool_sel, w_slab, b_slab,
             tiles_per_step=16):
    num_tiles = x_all.shape[0]
    T = int(tiles_per_step)
    grid = num_tiles // T
    TP = T * PAIRS

    tile32 = jnp.tile(jnp.eye(NN, dtype=jnp.bfloat16), (1, PAIRS))
    bmask = jnp.kron(jnp.eye(PAIRS, dtype=jnp.bfloat16),
                     jnp.ones((NN, NN), jnp.bfloat16))

    def tiled(shape):
        return pl.BlockSpec(shape, lambda i: (i,) + (0,) * (len(shape) - 1))

    def const(shape):
        return pl.BlockSpec(shape, lambda i: (0,) * len(shape))

    out = pl.pallas_call(
        _body,
        out_shape=jax.ShapeDtypeStruct((grid, 8, TP), jnp.float32),
        grid=(grid,),
        in_specs=[
            tiled((T, TILE_ROWS, D_IN)),
            tiled((T, PAIRS, NN, NN)),
            tiled((T, TILE_ROWS, 1)),
            tiled((T, POOL_ROWS, 1)),
            const((POOL_ROWS, TILE_ROWS)),
            const((_W_ROWS, HIDDEN)),
            const((_B_ROWS, HIDDEN)),
            const((NN, TILE_ROWS)),
            const((TILE_ROWS, TILE_ROWS)),
        ],
        out_specs=pl.BlockSpec((1, 8, TP), lambda i: (i, 0, 0)),
        scratch_shapes=[
            pltpu.VMEM((T * TILE_ROWS, HIDDEN), jnp.bfloat16),
            pltpu.VMEM((TP, HIDDEN), jnp.float32),
            pltpu.VMEM((TP, HIDDEN), jnp.float32),
        ],
        compiler_params=pltpu.CompilerParams(
            dimension_semantics=("parallel",)),
    )(x_all, a_cmp, mask, invn, pool_sel, w_slab, b_slab, tile32, bmask)

    return out[:, 0, :].reshape(-1, 1)


def kernel(x_all, a_cmp, mask, invn, pool_sel, w_slab, b_slab):
    return _forward(x_all, a_cmp, mask, invn, pool_sel, w_slab, b_slab)


# speedup vs baseline: 1.1512x; 1.1093x over previous
import jax
import jax.numpy as jnp
from jax import lax
from jax.experimental import pallas as pl
from jax.experimental.pallas import tpu as pltpu

MAX_NUMS = 16
NN = 2 * MAX_NUMS
D_IN = 8
HIDDEN = 32
PAIRS = 8
TILE_ROWS = PAIRS * NN
POOL_ROWS = 2 * PAIRS

_W0_OFF = 0
_W1_OFF = _W0_OFF + D_IN
_WF0A_OFF = _W1_OFF + HIDDEN
_WF0B_OFF = _WF0A_OFF + HIDDEN
_WF1_OFF = _WF0B_OFF + HIDDEN
_W_ROWS = 112
_B_ROWS = 8


def _body(x_ref, a_ref, mask_ref, invn_ref, pool_ref, w_ref, b_ref,
          tile32_ref, bmask_ref, out_ref, xw_sc, p1_sc, p2_sc):
    T = a_ref.shape[0]
    TP = p1_sc.shape[0]

    w = w_ref[...]
    w1 = w[_W1_OFF:_W1_OFF + HIDDEN, :]
    wf0a = w[_WF0A_OFF:_WF0A_OFF + HIDDEN, :]
    wf0b = w[_WF0B_OFF:_WF0B_OFF + HIDDEN, :]
    wf1r8 = w[_WF1_OFF:_WF1_OFF + 8, :]
    w0 = w[_W0_OFF:_W0_OFF + D_IN, :]
    b0 = b_ref[0:1, :]
    b1 = b_ref[1:2, :]
    bf0 = b_ref[2:3, :]
    bf1 = b_ref[3:4, 0:1]
    pool_sel = pool_ref[...]
    tile32 = tile32_ref[...]
    bmask = bmask_ref[...]

    x_flat = x_ref[...].reshape(T * TILE_ROWS, D_IN)
    xw_sc[...] = jnp.dot(x_flat, w0,
                         preferred_element_type=jnp.float32).astype(jnp.bfloat16)

    def tile_body(t, carry):
        a_rows = a_ref[t].reshape(TILE_ROWS, NN)
        bd = jnp.dot(a_rows, tile32,
                     preferred_element_type=jnp.float32)
        bd = bd.astype(jnp.bfloat16) * bmask

        r0 = pl.multiple_of(t * TILE_ROWS, TILE_ROWS)
        h = jnp.dot(bd, xw_sc[pl.ds(r0, TILE_ROWS), :],
                    preferred_element_type=jnp.float32) + b0
        h = jnp.maximum(h, 0.0)

        agg = jnp.dot(bd, h.astype(jnp.bfloat16),
                      preferred_element_type=jnp.float32)
        h = jnp.dot(agg.astype(jnp.bfloat16), w1,
                    preferred_element_type=jnp.float32) + b1
        h = jnp.maximum(h, 0.0)

        hm = (h * mask_ref[t]).astype(jnp.bfloat16)
        pooled = jnp.dot(pool_sel, hm,
                         preferred_element_type=jnp.float32)
        pooled = pooled * invn_ref[t]

        p0 = pl.multiple_of(t * PAIRS, PAIRS)
        p1_sc[pl.ds(p0, PAIRS), :] = pooled[:PAIRS, :]
        p2_sc[pl.ds(p0, PAIRS), :] = pooled[PAIRS:, :]
        return carry

    lax.fori_loop(0, T, tile_body, 0, unroll=4)

    hg1 = p1_sc[...].astype(jnp.bfloat16)
    hg2 = p2_sc[...].astype(jnp.bfloat16)
    z = (jnp.dot(hg1, wf0a, preferred_element_type=jnp.float32)
         + jnp.dot(hg2, wf0b, preferred_element_type=jnp.float32) + bf0)
    z = jnp.maximum(z, 0.0)
    logit8 = lax.dot_general(wf1r8, z.astype(jnp.bfloat16),
                             (((1,), (1,)), ((), ())),
                             preferred_element_type=jnp.float32)
    out_ref[...] = jax.nn.sigmoid(logit8 + bf1).reshape(1, 8, TP)


def _forward(x_all, a_cmp, mask, invn, pool_sel, w_slab, b_slab,
             tiles_per_step=16):
    num_tiles = x_all.shape[0]
    T = int(tiles_per_step)
    grid = num_tiles // T
    TP = T * PAIRS

    tile32 = jnp.tile(jnp.eye(NN, dtype=jnp.bfloat16), (1, PAIRS))
    bmask = jnp.kron(jnp.eye(PAIRS, dtype=jnp.bfloat16),
                     jnp.ones((NN, NN), jnp.bfloat16))

    def tiled(shape):
        return pl.BlockSpec(shape, lambda i: (i,) + (0,) * (len(shape) - 1))

    def const(shape):
        return pl.BlockSpec(shape, lambda i: (0,) * len(shape))

    out = pl.pallas_call(
        _body,
        out_shape=jax.ShapeDtypeStruct((grid, 8, TP), jnp.float32),
        grid=(grid,),
        in_specs=[
            tiled((T, TILE_ROWS, D_IN)),
            tiled((T, PAIRS, NN, NN)),
            tiled((T, TILE_ROWS, 1)),
            tiled((T, POOL_ROWS, 1)),
            const((POOL_ROWS, TILE_ROWS)),
            const((_W_ROWS, HIDDEN)),
            const((_B_ROWS, HIDDEN)),
            const((NN, TILE_ROWS)),
            const((TILE_ROWS, TILE_ROWS)),
        ],
        out_specs=pl.BlockSpec((1, 8, TP), lambda i: (i, 0, 0)),
        scratch_shapes=[
            pltpu.VMEM((T * TILE_ROWS, HIDDEN), jnp.bfloat16),
            pltpu.VMEM((TP, HIDDEN), jnp.float32),
            pltpu.VMEM((TP, HIDDEN), jnp.float32),
        ],
        compiler_params=pltpu.CompilerParams(
            dimension_semantics=("parallel",)),
    )(x_all, a_cmp, mask, invn, pool_sel, w_slab, b_slab, tile32, bmask)

    return out[:, 0, :].reshape(-1, 1)


def kernel(x_all, a_cmp, mask, invn, pool_sel, w_slab, b_slab):
    return _forward(x_all, a_cmp, mask, invn, pool_sel, w_slab, b_slab)


# speedup vs baseline: 2.5497x; 2.2149x over previous
import jax
import jax.numpy as jnp
from jax import lax
from jax.experimental import pallas as pl
from jax.experimental.pallas import tpu as pltpu

MAX_NUMS = 16
NN = 2 * MAX_NUMS
D_IN = 8
HIDDEN = 32
PAIRS = 8
TILE_ROWS = PAIRS * NN
POOL_ROWS = 2 * PAIRS

_W0_OFF = 0
_W1_OFF = _W0_OFF + D_IN
_WF0A_OFF = _W1_OFF + HIDDEN
_WF0B_OFF = _WF0A_OFF + HIDDEN
_WF1_OFF = _WF0B_OFF + HIDDEN
_W_ROWS = 112
_B_ROWS = 8

_C = (((0,), (0,)), ((), ()))
_TAB = (((0,), (1,)), ((), ()))


def _body(x_ref, a_ref, maskT_ref, invnT_ref, w_ref, bT_ref,
          tile32_ref, bmask_ref, pbig_ref, out_ref, g_sc, hm_sc):
    T = a_ref.shape[0]
    R = T * TILE_ROWS
    TP = T * PAIRS

    w = w_ref[...]
    w0 = w[_W0_OFF:_W0_OFF + D_IN, :]
    w1 = w[_W1_OFF:_W1_OFF + HIDDEN, :]
    wf0a = w[_WF0A_OFF:_WF0A_OFF + HIDDEN, :]
    wf0b = w[_WF0B_OFF:_WF0B_OFF + HIDDEN, :]
    wf1r8 = w[_WF1_OFF:_WF1_OFF + 8, :]
    bT = bT_ref[...]
    b0T = bT[:, 0:1]
    b1T = bT[:, 1:2]
    bf0T = bT[:, 2:3]
    bf1 = bT[0:1, 3:4]
    tile32 = tile32_ref[...]
    bmask = bmask_ref[...]

    x_flat = x_ref[...].reshape(R, D_IN)
    xwT = lax.dot_general(w0, x_flat, _TAB,
                          preferred_element_type=jnp.float32
                          ).astype(jnp.bfloat16)

    for t in range(T):
        lo, hi = t * TILE_ROWS, (t + 1) * TILE_ROWS
        a_rows = a_ref[t].reshape(TILE_ROWS, NN)
        bdT = lax.dot_general(tile32, a_rows, _TAB,
                              preferred_element_type=jnp.float32)
        bd = bdT.astype(jnp.bfloat16) * bmask

        agg0 = jnp.dot(xwT[:, lo:hi], bd,
                       preferred_element_type=jnp.float32)
        h0 = jnp.maximum(agg0 + b0T, 0.0).astype(jnp.bfloat16)
        agg1 = jnp.dot(h0, bd, preferred_element_type=jnp.float32)
        g_sc[:, lo:hi] = agg1.astype(jnp.bfloat16)

    h1 = lax.dot_general(w1, g_sc[...], _C,
                         preferred_element_type=jnp.float32) + b1T
    h1 = jnp.maximum(h1, 0.0)
    hm_sc[...] = (h1 * maskT_ref[0]).astype(jnp.bfloat16)

    pooled = jnp.dot(hm_sc[...], pbig_ref[...],
                     preferred_element_type=jnp.float32)
    pooled = pooled * invnT_ref[0]

    hg1 = pooled[:, :TP].astype(jnp.bfloat16)
    hg2 = pooled[:, TP:].astype(jnp.bfloat16)
    z = (lax.dot_general(wf0a, hg1, _C, preferred_element_type=jnp.float32)
         + lax.dot_general(wf0b, hg2, _C, preferred_element_type=jnp.float32)
         + bf0T)
    z = jnp.maximum(z, 0.0)
    logit8 = jnp.dot(wf1r8, z.astype(jnp.bfloat16),
                     preferred_element_type=jnp.float32)
    out_ref[...] = jax.nn.sigmoid(logit8 + bf1).reshape(1, 8, TP)


def _forward(x_all, a_cmp, mask, invn, pool_sel, w_slab, b_slab,
             tiles_per_step=16):
    del pool_sel
    num_tiles = x_all.shape[0]
    T = int(tiles_per_step)
    grid = num_tiles // T
    TP = T * PAIRS
    R = T * TILE_ROWS

    tile32 = jnp.tile(jnp.eye(NN, dtype=jnp.bfloat16), (1, PAIRS))
    bmask = jnp.kron(jnp.eye(PAIRS, dtype=jnp.bfloat16),
                     jnp.ones((NN, NN), jnp.bfloat16))
    bT = b_slab.T

    ridx = jnp.arange(R, dtype=jnp.int32)
    col = ((ridx % NN) // MAX_NUMS) * TP + (ridx // TILE_ROWS) * PAIRS \
        + (ridx % TILE_ROWS) // NN
    pbig = (col[:, None] == jnp.arange(2 * TP, dtype=jnp.int32)[None, :]
            ).astype(jnp.bfloat16)

    maskT = mask.reshape(grid, 1, R)
    invnT = invn.reshape(grid, T, 2, PAIRS).transpose(0, 2, 1, 3) \
        .reshape(grid, 1, 2 * TP)

    def tiled(shape):
        return pl.BlockSpec(shape, lambda i: (i,) + (0,) * (len(shape) - 1))

    def const(shape):
        return pl.BlockSpec(shape, lambda i: (0,) * len(shape))

    out = pl.pallas_call(
        _body,
        out_shape=jax.ShapeDtypeStruct((grid, 8, TP), jnp.float32),
        grid=(grid,),
        in_specs=[
            tiled((T, TILE_ROWS, D_IN)),
            tiled((T, PAIRS, NN, NN)),
            tiled((1, 1, R)),
            tiled((1, 1, 2 * TP)),
            const((_W_ROWS, HIDDEN)),
            const((HIDDEN, _B_ROWS)),
            const((NN, TILE_ROWS)),
            const((TILE_ROWS, TILE_ROWS)),
            const((R, 2 * TP)),
        ],
        out_specs=pl.BlockSpec((1, 8, TP), lambda i: (i, 0, 0)),
        scratch_shapes=[
            pltpu.VMEM((HIDDEN, R), jnp.bfloat16),
            pltpu.VMEM((HIDDEN, R), jnp.bfloat16),
        ],
        compiler_params=pltpu.CompilerParams(
            dimension_semantics=("parallel",)),
    )(x_all, a_cmp, maskT, invnT, w_slab, bT, tile32, bmask, pbig)

    return out[:, 0, :].reshape(-1, 1)


def kernel(x_all, a_cmp, mask, invn, pool_sel, w_slab, b_slab):
    return _forward(x_all, a_cmp, mask, invn, pool_sel, w_slab, b_slab)
